# Initial kernel scaffold; baseline (speedup 1.0000x reference)
import jax
import jax.numpy as jnp
from jax.experimental import pallas as pl


def kernel(state_buf, action_buf, next_state_buf, reward_buf, done_buf, keys):
    raise NotImplementedError("write your pallas kernel here")


# speedup vs baseline: 1.5588x; 1.5588x over previous
import functools

import jax
import jax.numpy as jnp
from jax import lax
from jax.experimental import pallas as pl
from jax.experimental.pallas import tpu as pltpu
from jax.experimental.pallas import tpu_sc as plsc

SIZE = 2048
B = 1024
D = 128 * 128

NC, NS = 2, 16
NW = NC * NS
BPW = B // NW
C = 2
NCHUNK = BPW // C

_mesh = plsc.VectorSubcoreMesh(core_axis_name="c", subcore_axis_name="s")


@functools.partial(
    pl.kernel,
    out_type=(
        jax.ShapeDtypeStruct((B, D), jnp.float32),
        jax.ShapeDtypeStruct((B, 4), jnp.float32),
        jax.ShapeDtypeStruct((B, D), jnp.float32),
        jax.ShapeDtypeStruct((B, 1), jnp.float32),
        jax.ShapeDtypeStruct((B, 1), jnp.float32),
    ),
    mesh=_mesh,
    compiler_params=pltpu.CompilerParams(use_tc_tiling_on_sc=False),
    scratch_types=[
        pltpu.VMEM((NCHUNK, C), jnp.int32),
        pltpu.VMEM((BPW,), jnp.int32),
        pltpu.VMEM((C, D), jnp.float32),
        pltpu.VMEM((BPW, 4), jnp.float32),
        pltpu.VMEM((BPW, 1), jnp.float32),
        pltpu.VMEM((BPW, 1), jnp.float32),
        pltpu.SemaphoreType.DMA,
    ],
)
def _gather_kernel(state_hbm, action_hbm, next_hbm, reward_hbm, done_hbm,
                   keys2_hbm, keys_hbm,
                   out_state, out_action, out_next, out_reward, out_done,
                   idx2, idxf, rowbuf, abuf, rbuf, dbuf, sem):
    wid = lax.axis_index("s") * NC + lax.axis_index("c")
    base = wid * BPW

    pltpu.sync_copy(keys2_hbm.at[pl.ds(wid * NCHUNK, NCHUNK)], idx2)
    pltpu.sync_copy(keys_hbm.at[pl.ds(base, BPW)], idxf)

    pltpu.async_copy(action_hbm.at[idxf], abuf, sem).wait()
    pltpu.sync_copy(abuf, out_action.at[pl.ds(base, BPW)])
    pltpu.async_copy(reward_hbm.at[idxf], rbuf, sem).wait()
    pltpu.sync_copy(rbuf, out_reward.at[pl.ds(base, BPW)])
    pltpu.async_copy(done_hbm.at[idxf], dbuf, sem).wait()
    pltpu.sync_copy(dbuf, out_done.at[pl.ds(base, BPW)])

    def big_phase(tab, out):
        def body(i, carry):
            pltpu.async_copy(tab.at[idx2.at[i]], rowbuf, sem).wait()
            pltpu.sync_copy(rowbuf, out.at[pl.ds(base + i * C, C)])
            return carry
        lax.fori_loop(0, NCHUNK, body, 0)

    big_phase(state_hbm, out_state)
    big_phase(next_hbm, out_next)


def kernel(state_buf, action_buf, next_state_buf, reward_buf, done_buf, keys):
    state2 = state_buf.reshape(SIZE, D)
    next2 = next_state_buf.reshape(SIZE, D)
    keys2 = keys.reshape(B // C, C)
    s, a, n, r, d = _gather_kernel(
        state2, action_buf, next2, reward_buf, done_buf, keys2, keys)
    return (s.reshape(B, 1, 128, 128), a, n.reshape(B, 1, 128, 128), r, d)


# speedup vs baseline: 1.6508x; 1.0590x over previous
import functools

import jax
import jax.numpy as jnp
from jax import lax
from jax.experimental import pallas as pl
from jax.experimental.pallas import tpu as pltpu
from jax.experimental.pallas import tpu_sc as plsc

SIZE = 2048
B = 1024
D = 128 * 128

NC, NS = 2, 16
NW = NC * NS
BPW = B // NW
C = 2
NCHUNK = BPW // C

_mesh = plsc.VectorSubcoreMesh(core_axis_name="c", subcore_axis_name="s")


@functools.partial(
    pl.kernel,
    out_type=(
        jax.ShapeDtypeStruct((B, D), jnp.float32),
        jax.ShapeDtypeStruct((B, D), jnp.float32),
        jax.ShapeDtypeStruct((B, 16), jnp.float32),
    ),
    mesh=_mesh,
    compiler_params=pltpu.CompilerParams(use_tc_tiling_on_sc=False),
    scratch_types=[
        pltpu.VMEM((NCHUNK, C), jnp.int32),
        pltpu.VMEM((BPW,), jnp.int32),
        pltpu.VMEM((C, D), jnp.float32),
        pltpu.VMEM((BPW, 16), jnp.float32),
        pltpu.SemaphoreType.DMA,
    ],
)
def _gather_kernel(state_hbm, next_hbm, comb_hbm, keys2_hbm, keys_hbm,
                   out_state, out_next, out_comb,
                   idx2, idxf, buf0, cbuf, sem):
    wid = lax.axis_index("s") * NC + lax.axis_index("c")
    base = wid * BPW

    pltpu.sync_copy(keys2_hbm.at[pl.ds(wid * NCHUNK, NCHUNK)], idx2)
    pltpu.sync_copy(keys_hbm.at[pl.ds(base, BPW)], idxf)

    pltpu.async_copy(comb_hbm.at[idxf], cbuf, sem).wait()
    pltpu.sync_copy(cbuf, out_comb.at[pl.ds(base, BPW)])

    def big_phase(tab, out):
        def gather(i, buf):
            return pltpu.make_async_copy(tab.at[idx2.at[i]], buf, sem)

        def write(i, buf):
            pltpu.sync_copy(buf, out.at[pl.ds(base + i * C, C)])

        def body(i, carry):
            pltpu.async_copy(tab.at[idx2.at[i]], buf0, sem).wait()
            pltpu.sync_copy(buf0, out.at[pl.ds(base + i * C, C)])
            return carry

        lax.fori_loop(0, NCHUNK, body, 0)

    big_phase(state_hbm, out_state)
    big_phase(next_hbm, out_next)


def kernel(state_buf, action_buf, next_state_buf, reward_buf, done_buf, keys):
    state2 = state_buf.reshape(SIZE, D)
    next2 = next_state_buf.reshape(SIZE, D)
    keys2 = keys.reshape(B // C, C)
    comb = jnp.concatenate(
        [action_buf, reward_buf, done_buf,
         jnp.zeros((SIZE, 10), jnp.float32)], axis=1)
    s, n, c = _gather_kernel(state2, next2, comb, keys2, keys)
    return (s.reshape(B, 1, 128, 128), c[:, :4],
            n.reshape(B, 1, 128, 128), c[:, 4:5], c[:, 5:6])


# speedup vs baseline: 1.8810x; 1.1394x over previous
import functools

import jax
import jax.numpy as jnp
from jax import lax
from jax.experimental import pallas as pl
from jax.experimental.pallas import tpu as pltpu
from jax.experimental.pallas import tpu_sc as plsc

SIZE = 2048
B = 1024
D = 128 * 128

NC, NS = 2, 16
NW = NC * NS
BPW = B // NW
C = 2
NCHUNK = BPW // C

_mesh = plsc.VectorSubcoreMesh(core_axis_name="c", subcore_axis_name="s")


@functools.partial(
    pl.kernel,
    out_type=(
        jax.ShapeDtypeStruct((B, D), jnp.float32),
        jax.ShapeDtypeStruct((B, D), jnp.float32),
        jax.ShapeDtypeStruct((B, 16), jnp.float32),
    ),
    mesh=_mesh,
    compiler_params=pltpu.CompilerParams(use_tc_tiling_on_sc=False),
    scratch_types=[
        pltpu.VMEM((NCHUNK, C), jnp.int32),
        pltpu.VMEM((BPW,), jnp.int32),
        pltpu.VMEM((C, D), jnp.float32),
        pltpu.VMEM((C, D), jnp.float32),
        pltpu.VMEM((BPW, 16), jnp.float32),
        pltpu.SemaphoreType.DMA,
    ],
)
def _gather_kernel(state_hbm, next_hbm, comb_hbm, keys2_hbm, keys_hbm,
                   out_state, out_next, out_comb,
                   idx2, idxf, buf0, buf1, cbuf, sem):
    wid = lax.axis_index("s") * NC + lax.axis_index("c")
    base = wid * BPW

    pltpu.sync_copy(keys2_hbm.at[pl.ds(wid * NCHUNK, NCHUNK)], idx2)
    pltpu.sync_copy(keys_hbm.at[pl.ds(base, BPW)], idxf)

    pltpu.async_copy(comb_hbm.at[idxf], cbuf, sem).wait()
    pltpu.sync_copy(cbuf, out_comb.at[pl.ds(base, BPW)])

    def big_phase(tab, out):
        def gather(i, buf):
            return pltpu.make_async_copy(tab.at[idx2.at[i]], buf, sem)

        def write(i, buf):
            pltpu.sync_copy(buf, out.at[pl.ds(base + i * C, C)])

        def body(j, carry):
            i = 2 * j
            gather(i, buf0).start()

            @pl.when(j >= 1)
            def _():
                write(i - 1, buf1)

            gather(i, buf0).wait()
            gather(i + 1, buf1).start()
            write(i, buf0)
            gather(i + 1, buf1).wait()
            return carry

        lax.fori_loop(0, NCHUNK // 2, body, 0)
        write(NCHUNK - 1, buf1)

    big_phase(state_hbm, out_state)
    big_phase(next_hbm, out_next)


def kernel(state_buf, action_buf, next_state_buf, reward_buf, done_buf, keys):
    state2 = state_buf.reshape(SIZE, D)
    next2 = next_state_buf.reshape(SIZE, D)
    keys2 = keys.reshape(B // C, C)
    comb = jnp.concatenate(
        [action_buf, reward_buf, done_buf,
         jnp.zeros((SIZE, 10), jnp.float32)], axis=1)
    s, n, c = _gather_kernel(state2, next2, comb, keys2, keys)
    return (s.reshape(B, 1, 128, 128), c[:, :4],
            n.reshape(B, 1, 128, 128), c[:, 4:5], c[:, 5:6])
